# baseline (speedup 1.0000x reference)
import functools

import jax
import jax.numpy as jnp
import numpy as np
from jax.experimental import pallas as pl
from jax.experimental.pallas import tpu as pltpu

N_HEADS = 16
N_KV = 8
HD = 128
ROPE_BASE = 500000.0

BT = 256
BN = 128
BQ = 256
BKV = 256


def _rope_block(x, pos_f):
    half = HD // 2
    k = jax.lax.broadcasted_iota(jnp.int32, (1, half), 1).astype(jnp.float32)
    inv_freq = jnp.exp(-(jnp.log(ROPE_BASE) / half) * k)
    ang = pos_f * inv_freq
    cos = jnp.cos(ang)
    sin = jnp.sin(ang)
    coscat = jnp.concatenate([cos, cos], axis=1)
    sincat = jnp.concatenate([-sin, sin], axis=1)
    swapped = jnp.concatenate([x[:, half:], x[:, :half]], axis=1)
    return x * coscat + swapped * sincat


def _qkv_body(h_ref, wv_ref, wl_ref, bv_ref, mask_ref, pos_ref, out_ref):
    j = pl.program_id(0)
    i = pl.program_id(1)
    bt = out_ref.shape[0]
    rows = h_ref[pl.ds(i * bt, bt), :].astype(jnp.bfloat16)
    mv = jnp.dot(rows, wv_ref[...].astype(jnp.bfloat16),
                 preferred_element_type=jnp.float32)
    mv = mv + bv_ref[...]
    ml = jnp.dot(rows, wl_ref[...].astype(jnp.bfloat16),
                 preferred_element_type=jnp.float32)
    mask = mask_ref[pl.ds(i * bt, bt), :] > 0
    mixed = jnp.where(mask, mv, ml)
    pos_f = pos_ref[pl.ds(i * bt, bt), :].astype(jnp.float32)
    roped = _rope_block(mixed, pos_f)
    out_ref[...] = jnp.where(j < N_HEADS + N_KV, roped, mixed)


def _attn_body(q_ref, k_ref, v_ref, o_ref, *, bq, bkv):
    qi = pl.program_id(1)
    q = q_ref[...].astype(jnp.bfloat16)
    nj = (qi * bq + bq + bkv - 1) // bkv
    scale = 1.0 / np.sqrt(HD)

    row_g = qi * bq + jax.lax.broadcasted_iota(jnp.int32, (bq, bkv), 0)

    def step(jj, carry):
        acc, m, l = carry
        kj = k_ref[pl.ds(jj * bkv, bkv), :].astype(jnp.bfloat16)
        vj = v_ref[pl.ds(jj * bkv, bkv), :].astype(jnp.bfloat16)
        s = jax.lax.dot_general(q, kj, (((1,), (1,)), ((), ())),
                                preferred_element_type=jnp.float32) * scale
        col_g = jj * bkv + jax.lax.broadcasted_iota(jnp.int32, (bq, bkv), 1)
        s = jnp.where(col_g <= row_g, s, -1e30)
        m_new = jnp.maximum(m, jnp.max(s, axis=1, keepdims=True))
        p = jnp.exp(s - m_new)
        alpha = jnp.exp(m - m_new)
        l_new = l * alpha + jnp.sum(p, axis=1, keepdims=True)
        acc_new = acc * alpha + jnp.dot(p.astype(jnp.bfloat16), vj,
                                        preferred_element_type=jnp.float32)
        return acc_new, m_new, l_new

    acc0 = jnp.zeros((bq, HD), jnp.float32)
    m0 = jnp.full((bq, 1), -1e30, jnp.float32)
    l0 = jnp.zeros((bq, 1), jnp.float32)
    acc, m, l = jax.lax.fori_loop(0, nj, step, (acc0, m0, l0))
    o_ref[...] = acc / l


def _dense_body(c_ref, wv_ref, wl_ref, mask_ref, out_ref):
    i = pl.program_id(1)
    bt = out_ref.shape[0]
    rows = c_ref[pl.ds(i * bt, bt), :].astype(jnp.bfloat16)
    ov = jnp.dot(rows, wv_ref[...].astype(jnp.bfloat16),
                 preferred_element_type=jnp.float32)
    ol = jnp.dot(rows, wl_ref[...].astype(jnp.bfloat16),
                 preferred_element_type=jnp.float32)
    mask = mask_ref[pl.ds(i * bt, bt), :] > 0
    out_ref[...] = jnp.where(mask, ov, ol)


def kernel(hidden_states, positions, vision_token_mask, Wv_qkv, bv_qkv,
           Wl_qkv, Wv_dense, Wl_dense):
    t, h = hidden_states.shape
    qkv = Wv_qkv.shape[1]
    d_out = Wv_dense.shape[1]
    mask2d = vision_token_mask.astype(jnp.int32).reshape(t, 1)
    pos2d = positions.astype(jnp.int32).reshape(t, 1)
    bv2d = bv_qkv.reshape(1, qkv)

    nj = qkv // BN
    ni = t // BT
    mixed = pl.pallas_call(
        _qkv_body,
        grid=(nj, ni),
        in_specs=[
            pl.BlockSpec((t, h), lambda j, i: (0, 0)),
            pl.BlockSpec((h, BN), lambda j, i: (0, j)),
            pl.BlockSpec((h, BN), lambda j, i: (0, j)),
            pl.BlockSpec((1, BN), lambda j, i: (0, j)),
            pl.BlockSpec((t, 1), lambda j, i: (0, 0)),
            pl.BlockSpec((t, 1), lambda j, i: (0, 0)),
        ],
        out_specs=pl.BlockSpec((BT, BN), lambda j, i: (i, j)),
        out_shape=jax.ShapeDtypeStruct((t, qkv), jnp.float32),
        compiler_params=pltpu.CompilerParams(
            dimension_semantics=("arbitrary", "arbitrary")),
    )(hidden_states, Wv_qkv, Wl_qkv, bv2d, mask2d, pos2d)

    nq = t // BQ
    ctx = pl.pallas_call(
        functools.partial(_attn_body, bq=BQ, bkv=BKV),
        grid=(N_HEADS, nq),
        in_specs=[
            pl.BlockSpec((BQ, HD), lambda hh, qi: (qi, hh)),
            pl.BlockSpec((t, HD), lambda hh, qi: (0, N_HEADS + hh // 2)),
            pl.BlockSpec((t, HD), lambda hh, qi: (0, N_HEADS + N_KV + hh // 2)),
        ],
        out_specs=pl.BlockSpec((BQ, HD), lambda hh, qi: (qi, hh)),
        out_shape=jax.ShapeDtypeStruct((t, N_HEADS * HD), jnp.float32),
        compiler_params=pltpu.CompilerParams(
            dimension_semantics=("arbitrary", "arbitrary")),
    )(mixed, mixed, mixed)

    nj2 = d_out // BN
    out = pl.pallas_call(
        _dense_body,
        grid=(nj2, ni),
        in_specs=[
            pl.BlockSpec((t, N_HEADS * HD), lambda j, i: (0, 0)),
            pl.BlockSpec((N_HEADS * HD, BN), lambda j, i: (0, j)),
            pl.BlockSpec((N_HEADS * HD, BN), lambda j, i: (0, j)),
            pl.BlockSpec((t, 1), lambda j, i: (0, 0)),
        ],
        out_specs=pl.BlockSpec((BT, BN), lambda j, i: (i, j)),
        out_shape=jax.ShapeDtypeStruct((t, d_out), jnp.float32),
        compiler_params=pltpu.CompilerParams(
            dimension_semantics=("arbitrary", "arbitrary")),
    )(ctx, Wv_dense, Wl_dense, mask2d)
    return out


# speedup vs baseline: 1.3133x; 1.3133x over previous
import functools

import jax
import jax.numpy as jnp
import numpy as np
from jax import lax
from jax.experimental import pallas as pl
from jax.experimental.pallas import tpu as pltpu

N_HEADS = 16
N_KV = 8
HD = 128
ROPE_BASE = 500000.0

BT = 256
BN = 256
BQ = 256
BKV = 256


def _rope_block(x, pos_f):
    half = HD // 2
    k = lax.broadcasted_iota(jnp.int32, (1, half), 1).astype(jnp.float32)
    inv_freq = jnp.exp(-(np.log(ROPE_BASE) / half) * k)
    ang = pos_f * inv_freq
    cos = jnp.cos(ang)
    sin = jnp.sin(ang)
    nh = x.shape[1] // HD
    coscat = jnp.concatenate([cos, cos] * nh, axis=1)
    sincat = jnp.concatenate([-sin, sin] * nh, axis=1)
    swapped = jnp.concatenate(
        sum(([x[:, c + half:c + HD], x[:, c:c + half]]
             for c in range(0, x.shape[1], HD)), []), axis=1)
    return x * coscat + swapped * sincat


def _qkv_body(h_ref, wv_ref, wl_ref, bv_ref, mask_ref, pos_ref, out_ref):
    j = pl.program_id(0)
    i = pl.program_id(1)
    bt = out_ref.shape[0]
    rows = h_ref[pl.ds(i * bt, bt), :]
    mv = jnp.dot(rows, wv_ref[...], preferred_element_type=jnp.float32)
    mv = mv + bv_ref[...]
    ml = jnp.dot(rows, wl_ref[...], preferred_element_type=jnp.float32)
    mask = mask_ref[pl.ds(i * bt, bt), :] > 0
    mixed = jnp.where(mask, mv, ml)
    pos_f = pos_ref[pl.ds(i * bt, bt), :].astype(jnp.float32)
    roped = _rope_block(mixed, pos_f)
    out_ref[...] = jnp.where(j < (N_HEADS + N_KV) * HD // out_ref.shape[1],
                             roped, mixed).astype(jnp.bfloat16)


def _attn_body(q_ref, k_ref, v_ref, o_ref, *, bq, bkv):
    qi = pl.program_id(1)
    q = q_ref[...]
    nj = (qi * bq + bq + bkv - 1) // bkv
    scale = 1.0 / np.sqrt(HD)
    row_g = qi * bq + lax.broadcasted_iota(jnp.int32, (bq, bkv), 0)

    def step(jj, carry):
        acc, m, l = carry
        kj = k_ref[pl.ds(jj * bkv, bkv), :]
        vj = v_ref[pl.ds(jj * bkv, bkv), :]
        s = lax.dot_general(q, kj, (((1,), (1,)), ((), ())),
                            preferred_element_type=jnp.float32) * scale
        col_g = jj * bkv + lax.broadcasted_iota(jnp.int32, (bq, bkv), 1)
        s = jnp.where(col_g <= row_g, s, -1e30)
        m_new = jnp.maximum(m, jnp.max(s, axis=1, keepdims=True))
        p = jnp.exp(s - m_new)
        alpha = jnp.exp(m - m_new)
        l_new = l * alpha + jnp.sum(p, axis=1, keepdims=True)
        acc_new = acc * alpha + jnp.dot(p.astype(jnp.bfloat16), vj,
                                        preferred_element_type=jnp.float32)
        return acc_new, m_new, l_new

    acc0 = jnp.zeros((bq, HD), jnp.float32)
    m0 = jnp.full((bq, 1), -1e30, jnp.float32)
    l0 = jnp.zeros((bq, 1), jnp.float32)
    acc, m, l = lax.fori_loop(0, nj, step, (acc0, m0, l0))
    o_ref[...] = (acc / l).astype(jnp.bfloat16)


def _dense_body(c_ref, wv_ref, wl_ref, mask_ref, out_ref):
    i = pl.program_id(1)
    bt = out_ref.shape[0]
    rows = c_ref[pl.ds(i * bt, bt), :]
    ov = jnp.dot(rows, wv_ref[...], preferred_element_type=jnp.float32)
    ol = jnp.dot(rows, wl_ref[...], preferred_element_type=jnp.float32)
    mask = mask_ref[pl.ds(i * bt, bt), :] > 0
    out_ref[...] = jnp.where(mask, ov, ol)


def kernel(hidden_states, positions, vision_token_mask, Wv_qkv, bv_qkv,
           Wl_qkv, Wv_dense, Wl_dense):
    t, h = hidden_states.shape
    qkv = Wv_qkv.shape[1]
    d_out = Wv_dense.shape[1]
    mask2d = vision_token_mask.astype(jnp.int32).reshape(t, 1)
    pos2d = positions.astype(jnp.int32).reshape(t, 1)
    bv2d = bv_qkv.reshape(1, qkv)
    h_bf = hidden_states.astype(jnp.bfloat16)
    wv_bf = Wv_qkv.astype(jnp.bfloat16)
    wl_bf = Wl_qkv.astype(jnp.bfloat16)
    wvd_bf = Wv_dense.astype(jnp.bfloat16)
    wld_bf = Wl_dense.astype(jnp.bfloat16)

    nj = qkv // BN
    ni = t // BT
    mixed = pl.pallas_call(
        _qkv_body,
        grid=(nj, ni),
        in_specs=[
            pl.BlockSpec((t, h), lambda j, i: (0, 0)),
            pl.BlockSpec((h, BN), lambda j, i: (0, j)),
            pl.BlockSpec((h, BN), lambda j, i: (0, j)),
            pl.BlockSpec((1, BN), lambda j, i: (0, j)),
            pl.BlockSpec((t, 1), lambda j, i: (0, 0)),
            pl.BlockSpec((t, 1), lambda j, i: (0, 0)),
        ],
        out_specs=pl.BlockSpec((BT, BN), lambda j, i: (i, j)),
        out_shape=jax.ShapeDtypeStruct((t, qkv), jnp.bfloat16),
        compiler_params=pltpu.CompilerParams(
            dimension_semantics=("arbitrary", "arbitrary")),
    )(h_bf, wv_bf, wl_bf, bv2d, mask2d, pos2d)

    nq = t // BQ
    ctx = pl.pallas_call(
        functools.partial(_attn_body, bq=BQ, bkv=BKV),
        grid=(N_HEADS, nq),
        in_specs=[
            pl.BlockSpec((BQ, HD), lambda hh, qi: (qi, hh)),
            pl.BlockSpec((t, HD), lambda hh, qi: (0, N_HEADS + hh // 2)),
            pl.BlockSpec((t, HD), lambda hh, qi: (0, N_HEADS + N_KV + hh // 2)),
        ],
        out_specs=pl.BlockSpec((BQ, HD), lambda hh, qi: (qi, hh)),
        out_shape=jax.ShapeDtypeStruct((t, N_HEADS * HD), jnp.bfloat16),
        compiler_params=pltpu.CompilerParams(
            dimension_semantics=("arbitrary", "arbitrary")),
    )(mixed, mixed, mixed)

    nj2 = d_out // BN
    out = pl.pallas_call(
        _dense_body,
        grid=(nj2, ni),
        in_specs=[
            pl.BlockSpec((t, N_HEADS * HD), lambda j, i: (0, 0)),
            pl.BlockSpec((N_HEADS * HD, BN), lambda j, i: (0, j)),
            pl.BlockSpec((N_HEADS * HD, BN), lambda j, i: (0, j)),
            pl.BlockSpec((t, 1), lambda j, i: (0, 0)),
        ],
        out_specs=pl.BlockSpec((BT, BN), lambda j, i: (i, j)),
        out_shape=jax.ShapeDtypeStruct((t, d_out), jnp.float32),
        compiler_params=pltpu.CompilerParams(
            dimension_semantics=("arbitrary", "arbitrary")),
    )(ctx, wvd_bf, wld_bf, mask2d)
    return out


# speedup vs baseline: 1.3297x; 1.0125x over previous
import functools

import jax
import jax.numpy as jnp
import numpy as np
from jax import lax
from jax.experimental import pallas as pl
from jax.experimental.pallas import tpu as pltpu

N_HEADS = 16
N_KV = 8
HD = 128
ROPE_BASE = 500000.0

BT = 256
BN = 256
BQ = 256
BKV = 256


def _rope_block(x, pos_f):
    half = HD // 2
    k = lax.broadcasted_iota(jnp.int32, (1, half), 1).astype(jnp.float32)
    inv_freq = jnp.exp(-(np.log(ROPE_BASE) / half) * k)
    ang = pos_f * inv_freq
    cos = jnp.cos(ang)
    sin = jnp.sin(ang)
    nh = x.shape[1] // HD
    coscat = jnp.concatenate([cos, cos] * nh, axis=1)
    sincat = jnp.concatenate([-sin, sin] * nh, axis=1)
    swapped = jnp.concatenate(
        sum(([x[:, c + half:c + HD], x[:, c:c + half]]
             for c in range(0, x.shape[1], HD)), []), axis=1)
    return x * coscat + swapped * sincat


def _qkv_body(h_ref, wv_ref, wl_ref, bv_ref, mask_ref, pos_ref, out_ref):
    j = pl.program_id(0)
    i = pl.program_id(1)
    bt = out_ref.shape[0]
    rows = h_ref[pl.ds(i * bt, bt), :]
    mv = jnp.dot(rows, wv_ref[...], preferred_element_type=jnp.float32)
    mv = mv + bv_ref[...]
    ml = jnp.dot(rows, wl_ref[...], preferred_element_type=jnp.float32)
    mask = mask_ref[pl.ds(i * bt, bt), :] > 0
    mixed = jnp.where(mask, mv, ml)
    pos_f = pos_ref[pl.ds(i * bt, bt), :].astype(jnp.float32)
    roped = _rope_block(mixed, pos_f)
    out_ref[...] = jnp.where(j < (N_HEADS + N_KV) * HD // out_ref.shape[1],
                             roped, mixed).astype(jnp.bfloat16)


def _attn_body(q_ref, k_ref, v_ref, o_ref, *, bq, bkv):
    qi = pl.program_id(1)
    scale = 1.0 / np.sqrt(HD)
    q = (q_ref[...].astype(jnp.float32) * scale).astype(jnp.bfloat16)

    def chunk(jj, carry, masked):
        acc, m, l = carry
        kj = k_ref[pl.ds(jj * bkv, bkv), :]
        vj = v_ref[pl.ds(jj * bkv, bkv), :]
        s = lax.dot_general(q, kj, (((1,), (1,)), ((), ())),
                            preferred_element_type=jnp.float32)
        if masked:
            row_l = lax.broadcasted_iota(jnp.int32, (bq, bkv), 0)
            col_l = lax.broadcasted_iota(jnp.int32, (bq, bkv), 1)
            s = jnp.where(col_l <= row_l, s, -1e30)
        m_new = jnp.maximum(m, jnp.max(s, axis=1, keepdims=True))
        p = jnp.exp(s - m_new)
        alpha = jnp.exp(m - m_new)
        l_new = l * alpha + jnp.sum(p, axis=1, keepdims=True)
        acc_new = acc * alpha + jnp.dot(p.astype(jnp.bfloat16), vj,
                                        preferred_element_type=jnp.float32)
        return acc_new, m_new, l_new

    acc0 = jnp.zeros((bq, HD), jnp.float32)
    m0 = jnp.full((bq, 1), -1e30, jnp.float32)
    l0 = jnp.zeros((bq, 1), jnp.float32)
    carry = lax.fori_loop(0, qi, lambda jj, c: chunk(jj, c, False),
                          (acc0, m0, l0))
    acc, m, l = chunk(qi, carry, True)
    o_ref[...] = (acc / l).astype(jnp.bfloat16)


def _dense_body(c_ref, wv_ref, wl_ref, mask_ref, out_ref):
    i = pl.program_id(1)
    bt = out_ref.shape[0]
    rows = c_ref[pl.ds(i * bt, bt), :]
    ov = jnp.dot(rows, wv_ref[...], preferred_element_type=jnp.float32)
    ol = jnp.dot(rows, wl_ref[...], preferred_element_type=jnp.float32)
    mask = mask_ref[pl.ds(i * bt, bt), :] > 0
    out_ref[...] = jnp.where(mask, ov, ol)


def kernel(hidden_states, positions, vision_token_mask, Wv_qkv, bv_qkv,
           Wl_qkv, Wv_dense, Wl_dense):
    t, h = hidden_states.shape
    qkv = Wv_qkv.shape[1]
    d_out = Wv_dense.shape[1]
    mask2d = vision_token_mask.astype(jnp.int32).reshape(t, 1)
    pos2d = positions.astype(jnp.int32).reshape(t, 1)
    bv2d = bv_qkv.reshape(1, qkv)
    h_bf = hidden_states.astype(jnp.bfloat16)
    wv_bf = Wv_qkv.astype(jnp.bfloat16)
    wl_bf = Wl_qkv.astype(jnp.bfloat16)
    wvd_bf = Wv_dense.astype(jnp.bfloat16)
    wld_bf = Wl_dense.astype(jnp.bfloat16)

    nj = qkv // BN
    ni = t // BT
    mixed = pl.pallas_call(
        _qkv_body,
        grid=(nj, ni),
        in_specs=[
            pl.BlockSpec((t, h), lambda j, i: (0, 0)),
            pl.BlockSpec((h, BN), lambda j, i: (0, j)),
            pl.BlockSpec((h, BN), lambda j, i: (0, j)),
            pl.BlockSpec((1, BN), lambda j, i: (0, j)),
            pl.BlockSpec((t, 1), lambda j, i: (0, 0)),
            pl.BlockSpec((t, 1), lambda j, i: (0, 0)),
        ],
        out_specs=pl.BlockSpec((BT, BN), lambda j, i: (i, j)),
        out_shape=jax.ShapeDtypeStruct((t, qkv), jnp.bfloat16),
        compiler_params=pltpu.CompilerParams(
            dimension_semantics=("arbitrary", "arbitrary")),
    )(h_bf, wv_bf, wl_bf, bv2d, mask2d, pos2d)

    nq = t // BQ
    ctx = pl.pallas_call(
        functools.partial(_attn_body, bq=BQ, bkv=BKV),
        grid=(N_HEADS, nq),
        in_specs=[
            pl.BlockSpec((BQ, HD), lambda hh, qi: (qi, hh)),
            pl.BlockSpec((t, HD), lambda hh, qi: (0, N_HEADS + hh // 2)),
            pl.BlockSpec((t, HD), lambda hh, qi: (0, N_HEADS + N_KV + hh // 2)),
        ],
        out_specs=pl.BlockSpec((BQ, HD), lambda hh, qi: (qi, hh)),
        out_shape=jax.ShapeDtypeStruct((t, N_HEADS * HD), jnp.bfloat16),
        compiler_params=pltpu.CompilerParams(
            dimension_semantics=("arbitrary", "arbitrary")),
    )(mixed, mixed, mixed)

    nj2 = d_out // BN
    out = pl.pallas_call(
        _dense_body,
        grid=(nj2, ni),
        in_specs=[
            pl.BlockSpec((t, N_HEADS * HD), lambda j, i: (0, 0)),
            pl.BlockSpec((N_HEADS * HD, BN), lambda j, i: (0, j)),
            pl.BlockSpec((N_HEADS * HD, BN), lambda j, i: (0, j)),
            pl.BlockSpec((t, 1), lambda j, i: (0, 0)),
        ],
        out_specs=pl.BlockSpec((BT, BN), lambda j, i: (i, j)),
        out_shape=jax.ShapeDtypeStruct((t, d_out), jnp.float32),
        compiler_params=pltpu.CompilerParams(
            dimension_semantics=("arbitrary", "arbitrary")),
    )(ctx, wvd_bf, wld_bf, mask2d)
    return out
